# baseline (speedup 1.0000x reference)
import jax
import jax.numpy as jnp
from jax import lax
from jax.experimental import pallas as pl
from jax.experimental.pallas import tpu as pltpu
from jax.experimental.pallas import tpu_sc as plsc

D = 64
NC, NS = 2, 16
NW = NC * NS
C = 128
SCALE = 8.0


def _emb_body(x_hbm, table_hbm, out_hbm, idx_v, gbufs, obufs, gsems, osems):
    nbuf = len(gbufs)
    nch = x_hbm.shape[1]
    wid = lax.axis_index("s") * NC + lax.axis_index("c")
    pltpu.sync_copy(x_hbm.at[wid], idx_v)

    for b in range(nbuf):
        pltpu.async_copy(table_hbm.at[idx_v.at[b]], gbufs[b], gsems[b])

    out_base = wid * (nch * C)

    @pl.loop(0, nch, step=nbuf)
    def _chunk(g):
        for b in range(nbuf):
            j = g + b
            pltpu.make_async_copy(
                table_hbm.at[idx_v.at[j]], gbufs[b], gsems[b]).wait()

            @pl.when(j >= nbuf)
            def _():
                pltpu.make_async_copy(
                    obufs[b],
                    out_hbm.at[pl.ds(out_base + (j - nbuf) * C, C)],
                    osems[b]).wait()

            @pl.loop(0, C, step=4)
            def _row(i):
                for u in range(4):
                    for c in range(4):
                        sl = pl.ds(c * 16, 16)
                        obufs[b][i + u, sl] = gbufs[b][i + u, sl] * SCALE

            @pl.when(j + nbuf < nch)
            def _():
                pltpu.async_copy(
                    table_hbm.at[idx_v.at[j + nbuf]], gbufs[b], gsems[b])

            pltpu.async_copy(
                obufs[b], out_hbm.at[pl.ds(out_base + j * C, C)], osems[b])

    for b in range(nbuf):
        pltpu.make_async_copy(
            obufs[b],
            out_hbm.at[pl.ds(out_base + (nch - nbuf + b) * C, C)],
            osems[b]).wait()


def kernel(x, table):
    B, S = x.shape
    n_idx = B * S
    assert n_idx % (NW * C) == 0
    nch = n_idx // (NW * C)
    x_r = x.reshape(NW, nch, C).astype(jnp.int32)

    nbuf = 4
    mesh = plsc.VectorSubcoreMesh(core_axis_name="c", subcore_axis_name="s")
    k = pl.kernel(
        _emb_body,
        out_type=jax.ShapeDtypeStruct((n_idx, D), jnp.float32),
        mesh=mesh,
        compiler_params=pltpu.CompilerParams(use_tc_tiling_on_sc=False),
        scratch_types=[
            pltpu.VMEM((nch, C), jnp.int32),
            [pltpu.VMEM((C, D), jnp.float32) for _ in range(nbuf)],
            [pltpu.VMEM((C, D), jnp.float32) for _ in range(nbuf)],
            [pltpu.SemaphoreType.DMA for _ in range(nbuf)],
            [pltpu.SemaphoreType.DMA for _ in range(nbuf)],
        ],
    )
    out = k(x_r, table)
    return out.reshape(B, S, D)
